# initial kernel scaffold (unmeasured)
import jax
import jax.numpy as jnp
from jax import lax
from jax.experimental import pallas as pl
from jax.experimental.pallas import tpu as pltpu

N_DEV = 8
B, SQ, SKV, E = 2, 512, 512, 768
HQ, DH = 64, 64
H_LOC = HQ // N_DEV
BLK = 64


def kernel(x, Wq, K_ext, V_ext, Wo):
    def body(x_ref, wq_ref, k_ref, v_ref, wo_ref, out_ref,
             comm_ref, send_sems, recv_sems, credit_sem):
        my = lax.axis_index("i")
        left = lax.rem(my + N_DEV - 1, N_DEV)
        right = lax.rem(my + 1, N_DEV)

        barrier_sem = pltpu.get_barrier_semaphore()
        for nbr in (left, right):
            pl.semaphore_signal(barrier_sem, inc=1, device_id=(nbr,),
                                device_id_type=pl.DeviceIdType.MESH)
        pl.semaphore_wait(barrier_sem, 2)

        x2 = x_ref[...].reshape(B * SQ, E)
        q2 = jnp.dot(x2, wq_ref[...], preferred_element_type=jnp.float32)

        rows = lax.broadcasted_iota(jnp.int32, (SQ, SKV), 0) // BLK
        cols = lax.broadcasted_iota(jnp.int32, (SQ, SKV), 1) // BLK
        mask = (rows == cols) | ((cols % 4) == (rows % 4))
        neg = jnp.float32(-1e9)

        h0 = my * H_LOC
        parts = []
        for b in range(B):
            kb = k_ref[b, :, pl.ds(h0, H_LOC), :]
            vb = v_ref[b, :, pl.ds(h0, H_LOC), :]
            qb = q2[b * SQ:(b + 1) * SQ, :]
            ctx_cols = []
            for h in range(H_LOC):
                q = qb[:, h * DH:(h + 1) * DH]
                k = kb[:, h, :]
                v = vb[:, h, :]
                s = lax.dot_general(q, k, (((1,), (1,)), ((), ())),
                                    preferred_element_type=jnp.float32)
                s = s * 0.125
                s = jnp.where(mask, s, neg)
                m = jnp.max(s, axis=1, keepdims=True)
                w = jnp.exp(s - m)
                w = w / jnp.sum(w, axis=1, keepdims=True)
                ctx_cols.append(jnp.dot(w, v, preferred_element_type=jnp.float32))
            ctx = jnp.concatenate(ctx_cols, axis=1)
            parts.append(jnp.dot(ctx, wo_ref[...],
                                 preferred_element_type=jnp.float32))
        partial = jnp.concatenate(parts, axis=0)

        comm_ref[0] = partial
        out_ref[...] = partial.reshape(B, SQ, E)

        for h in range(N_DEV - 1):
            s_slot = h % 2
            r_slot = (h + 1) % 2
            if h >= 1:
                pl.semaphore_wait(credit_sem, 1)
            rdma = pltpu.make_async_remote_copy(
                src_ref=comm_ref.at[s_slot],
                dst_ref=comm_ref.at[r_slot],
                send_sem=send_sems.at[s_slot],
                recv_sem=recv_sems.at[r_slot],
                device_id=(right,),
                device_id_type=pl.DeviceIdType.MESH,
            )
            rdma.start()
            rdma.wait()
            if h < N_DEV - 2:
                pl.semaphore_signal(credit_sem, inc=1, device_id=(left,),
                                    device_id_type=pl.DeviceIdType.MESH)
            out_ref[...] = out_ref[...] + comm_ref[r_slot].reshape(B, SQ, E)

    return pl.pallas_call(
        body,
        out_shape=jax.ShapeDtypeStruct((B, SQ, E), jnp.float32),
        in_specs=[pl.BlockSpec(memory_space=pltpu.VMEM)] * 5,
        out_specs=pl.BlockSpec(memory_space=pltpu.VMEM),
        scratch_shapes=[
            pltpu.VMEM((2, B * SQ, E), jnp.float32),
            pltpu.SemaphoreType.DMA((2,)),
            pltpu.SemaphoreType.DMA((2,)),
            pltpu.SemaphoreType.REGULAR,
        ],
        compiler_params=pltpu.CompilerParams(collective_id=0),
    )(x, Wq, K_ext, V_ext, Wo)


# baseline (device time: 339784 ns/iter reference)
import jax
import jax.numpy as jnp
from jax import lax
from jax.experimental import pallas as pl
from jax.experimental.pallas import tpu as pltpu

N_DEV = 8
B, SQ, SKV, E = 2, 512, 512, 768
HQ, DH = 64, 64
H_LOC = HQ // N_DEV
BLK = 64


def kernel(x, Wq, K_ext, V_ext, Wo):
    def body(x_ref, wq_ref, k_hbm, v_hbm, wo_ref, out_ref,
             comm_ref, k_loc, v_loc, copy_sems, send_sems, recv_sems,
             credit_sem):
        my = lax.axis_index("i")
        left = lax.rem(my + N_DEV - 1, N_DEV)
        right = lax.rem(my + 1, N_DEV)

        h0 = my * H_LOC
        kcp = pltpu.make_async_copy(
            k_hbm.at[:, :, pl.ds(h0, H_LOC), :], k_loc, copy_sems.at[0])
        vcp = pltpu.make_async_copy(
            v_hbm.at[:, :, pl.ds(h0, H_LOC), :], v_loc, copy_sems.at[1])
        kcp.start()
        vcp.start()

        barrier_sem = pltpu.get_barrier_semaphore()
        for nbr in (left, right):
            pl.semaphore_signal(barrier_sem, inc=1, device_id=(nbr,),
                                device_id_type=pl.DeviceIdType.MESH)
        pl.semaphore_wait(barrier_sem, 2)

        x2 = x_ref[...].reshape(B * SQ, E)
        q2 = jnp.dot(x2, wq_ref[...], preferred_element_type=jnp.float32)

        rows = lax.broadcasted_iota(jnp.int32, (SQ, SKV), 0) // BLK
        cols = lax.broadcasted_iota(jnp.int32, (SQ, SKV), 1) // BLK
        mask = (rows == cols) | ((cols % 4) == (rows % 4))
        neg = jnp.float32(-1e9)

        kcp.wait()
        vcp.wait()
        parts = []
        for b in range(B):
            kb = k_loc[b]
            vb = v_loc[b]
            qb = q2[b * SQ:(b + 1) * SQ, :]
            ctx_cols = []
            for h in range(H_LOC):
                q = qb[:, h * DH:(h + 1) * DH]
                k = kb[:, h, :]
                v = vb[:, h, :]
                s = lax.dot_general(q, k, (((1,), (1,)), ((), ())),
                                    preferred_element_type=jnp.float32)
                s = s * 0.125
                s = jnp.where(mask, s, neg)
                m = jnp.max(s, axis=1, keepdims=True)
                w = jnp.exp(s - m)
                w = w / jnp.sum(w, axis=1, keepdims=True)
                ctx_cols.append(jnp.dot(w, v, preferred_element_type=jnp.float32))
            ctx = jnp.concatenate(ctx_cols, axis=1)
            parts.append(jnp.dot(ctx, wo_ref[...],
                                 preferred_element_type=jnp.float32))
        partial = jnp.concatenate(parts, axis=0)

        comm_ref[0] = partial
        out_ref[...] = partial.reshape(B, SQ, E)

        for h in range(N_DEV - 1):
            s_slot = h % 2
            r_slot = (h + 1) % 2
            if h >= 1:
                pl.semaphore_wait(credit_sem, 1)
            rdma = pltpu.make_async_remote_copy(
                src_ref=comm_ref.at[s_slot],
                dst_ref=comm_ref.at[r_slot],
                send_sem=send_sems.at[s_slot],
                recv_sem=recv_sems.at[r_slot],
                device_id=(right,),
                device_id_type=pl.DeviceIdType.MESH,
            )
            rdma.start()
            rdma.wait()
            if h < N_DEV - 2:
                pl.semaphore_signal(credit_sem, inc=1, device_id=(left,),
                                    device_id_type=pl.DeviceIdType.MESH)
            out_ref[...] = out_ref[...] + comm_ref[r_slot].reshape(B, SQ, E)

    return pl.pallas_call(
        body,
        out_shape=jax.ShapeDtypeStruct((B, SQ, E), jnp.float32),
        in_specs=[
            pl.BlockSpec(memory_space=pltpu.VMEM),
            pl.BlockSpec(memory_space=pltpu.VMEM),
            pl.BlockSpec(memory_space=pl.ANY),
            pl.BlockSpec(memory_space=pl.ANY),
            pl.BlockSpec(memory_space=pltpu.VMEM),
        ],
        out_specs=pl.BlockSpec(memory_space=pltpu.VMEM),
        scratch_shapes=[
            pltpu.VMEM((2, B * SQ, E), jnp.float32),
            pltpu.VMEM((B, SKV, H_LOC, DH), jnp.float32),
            pltpu.VMEM((B, SKV, H_LOC, DH), jnp.float32),
            pltpu.SemaphoreType.DMA((2,)),
            pltpu.SemaphoreType.DMA((2,)),
            pltpu.SemaphoreType.DMA((2,)),
            pltpu.SemaphoreType.REGULAR,
        ],
        compiler_params=pltpu.CompilerParams(collective_id=0),
    )(x, Wq, K_ext, V_ext, Wo)


# device time: 113568 ns/iter; 2.9919x vs baseline; 2.9919x over previous
import jax
import jax.numpy as jnp
from jax import lax
from jax.experimental import pallas as pl
from jax.experimental.pallas import tpu as pltpu

N_DEV = 8
B, SQ, SKV, E = 2, 512, 512, 768
HQ, DH = 64, 64
H_LOC = HQ // N_DEV
BLK = 64
R = B * SQ


def kernel(x, Wq, K_ext, V_ext, Wo):
    def body(x_ref, wq_ref, k_hbm, v_hbm, wo_ref, out_ref,
             cbuf, rbuf, k_loc, v_loc, q2_ref, copy_sems, send_sems,
             recv_sems):
        my = lax.axis_index("i")

        h0 = my * H_LOC
        kcp = pltpu.make_async_copy(
            k_hbm.at[:, :, pl.ds(h0, H_LOC), :], k_loc, copy_sems.at[0])
        vcp = pltpu.make_async_copy(
            v_hbm.at[:, :, pl.ds(h0, H_LOC), :], v_loc, copy_sems.at[1])
        kcp.start()
        vcp.start()

        r = my ^ ((my >> 1) & 1)
        r0 = r & 1
        r1 = (r >> 1) & 1
        r2 = (r >> 2) & 1

        def pos_of(q):
            return q ^ ((q >> 1) & 1)

        px = pos_of(r ^ 1)
        py = pos_of(r ^ 2)
        pz = pos_of(r ^ 4)

        barrier_sem = pltpu.get_barrier_semaphore()
        for nbr in (px, py, pz):
            pl.semaphore_signal(barrier_sem, inc=1, device_id=(nbr,),
                                device_id_type=pl.DeviceIdType.MESH)
        pl.semaphore_wait(barrier_sem, 3)

        k1 = SQ * r0
        o1 = SQ * (1 - r0)
        k2 = k1 + 256 * r1
        o2 = k1 + 256 * (1 - r1)
        k3 = k2 + 128 * r2
        o3 = k2 + 128 * (1 - r2)

        def exchange(idx, src_region, dst_is_rbuf, dst_region, partner):
            dst = rbuf if dst_is_rbuf else cbuf
            return pltpu.make_async_remote_copy(
                src_ref=cbuf.at[src_region],
                dst_ref=dst.at[dst_region],
                send_sem=send_sems.at[idx],
                recv_sem=recv_sems.at[idx],
                device_id=(partner,),
                device_id_type=pl.DeviceIdType.MESH,
            )

        xb = x_ref[...].reshape(R, E).astype(jnp.bfloat16)
        wq = wq_ref[...].astype(jnp.bfloat16)
        wo = wo_ref[...].astype(jnp.bfloat16)
        q2_ref[...] = jnp.dot(xb, wq,
                              preferred_element_type=jnp.float32)

        rows = lax.broadcasted_iota(jnp.int32, (SQ, SKV), 0) // BLK
        cols = lax.broadcasted_iota(jnp.int32, (SQ, SKV), 1) // BLK
        live = (rows == cols) | ((cols % 4) == (rows % 4))
        bias = jnp.where(live, jnp.float32(0.0), jnp.float32(-30.0))

        kcp.wait()
        vcp.wait()

        def compute_batch(b):
            qb = q2_ref[pl.ds(b * SQ, SQ), :]
            kb = k_loc[pl.ds(b, 1)][0].astype(jnp.bfloat16)
            vb = v_loc[pl.ds(b, 1)][0].astype(jnp.bfloat16)
            ctx_cols = []
            for h in range(H_LOC):
                q = qb[:, h * DH:(h + 1) * DH].astype(jnp.bfloat16)
                k = kb[:, h, :]
                v = vb[:, h, :]
                s = lax.dot_general(q, k, (((1,), (1,)), ((), ())),
                                    preferred_element_type=jnp.float32)
                s = s * 0.125 + bias
                w = jnp.exp(s)
                denom = jnp.sum(w, axis=1, keepdims=True)
                ctx_h = jnp.dot(w.astype(jnp.bfloat16), v,
                                preferred_element_type=jnp.float32)
                ctx_cols.append(ctx_h / denom)
            ctx = jnp.concatenate(ctx_cols, axis=1).astype(jnp.bfloat16)
            return jnp.dot(ctx, wo, preferred_element_type=jnp.float32)

        part_send = compute_batch(1 - r0)
        cbuf[pl.ds(o1, SQ)] = part_send.astype(jnp.bfloat16)
        rs1 = exchange(0, pl.ds(o1, SQ), True, pl.ds(0, 512), px)
        rs1.start()

        part_keep = compute_batch(r0)
        rs1.wait()
        out_ref[pl.ds(k1, SQ)] = part_keep + rbuf[0:512].astype(jnp.float32)

        cbuf[pl.ds(o2, 256)] = out_ref[pl.ds(o2, 256)].astype(jnp.bfloat16)
        rs2 = exchange(1, pl.ds(o2, 256), True, pl.ds(512, 256), py)
        rs2.start()
        rs2.wait()
        out_ref[pl.ds(k2, 256)] = (out_ref[pl.ds(k2, 256)]
                                   + rbuf[512:768].astype(jnp.float32))

        cbuf[pl.ds(o3, 128)] = out_ref[pl.ds(o3, 128)].astype(jnp.bfloat16)
        rs3 = exchange(2, pl.ds(o3, 128), True, pl.ds(768, 128), pz)
        rs3.start()
        rs3.wait()
        out_ref[pl.ds(k3, 128)] = (out_ref[pl.ds(k3, 128)]
                                   + rbuf[768:896].astype(jnp.float32))

        cbuf[pl.ds(k3, 128)] = out_ref[pl.ds(k3, 128)].astype(jnp.bfloat16)
        ag1 = exchange(3, pl.ds(k3, 128), False, pl.ds(k3, 128), pz)
        ag1.start()
        ag1.wait()

        ag2 = exchange(4, pl.ds(k2, 256), False, pl.ds(k2, 256), py)
        ag2.start()
        ag2.wait()

        ag3 = exchange(5, pl.ds(k1, 512), False, pl.ds(k1, 512), px)
        ag3.start()
        ag3.wait()

        out_ref[pl.ds(o3, 128)] = cbuf[pl.ds(o3, 128)].astype(jnp.float32)
        out_ref[pl.ds(o2, 256)] = cbuf[pl.ds(o2, 256)].astype(jnp.float32)
        out_ref[pl.ds(o1, 512)] = cbuf[pl.ds(o1, 512)].astype(jnp.float32)

    out = pl.pallas_call(
        body,
        out_shape=jax.ShapeDtypeStruct((R, E), jnp.float32),
        in_specs=[
            pl.BlockSpec(memory_space=pltpu.VMEM),
            pl.BlockSpec(memory_space=pltpu.VMEM),
            pl.BlockSpec(memory_space=pl.ANY),
            pl.BlockSpec(memory_space=pl.ANY),
            pl.BlockSpec(memory_space=pltpu.VMEM),
        ],
        out_specs=pl.BlockSpec(memory_space=pltpu.VMEM),
        scratch_shapes=[
            pltpu.VMEM((R, E), jnp.bfloat16),
            pltpu.VMEM((896, E), jnp.bfloat16),
            pltpu.VMEM((B, SKV, H_LOC, DH), jnp.float32),
            pltpu.VMEM((B, SKV, H_LOC, DH), jnp.float32),
            pltpu.VMEM((R, H_LOC * DH), jnp.float32),
            pltpu.SemaphoreType.DMA((2,)),
            pltpu.SemaphoreType.DMA((6,)),
            pltpu.SemaphoreType.DMA((6,)),
        ],
        compiler_params=pltpu.CompilerParams(collective_id=0),
    )(x, Wq, K_ext, V_ext, Wo)
    return out.reshape(B, SQ, E)


# device time: 61220 ns/iter; 5.5502x vs baseline; 1.8551x over previous
import os

import jax
import jax.numpy as jnp
from jax import lax
from jax.experimental import pallas as pl
from jax.experimental.pallas import tpu as pltpu

N_DEV = 8
B, SQ, SKV, E = 2, 512, 512, 768
HQ, DH = 64, 64
H_LOC = HQ // N_DEV
BLK = 64
R = B * SQ


def kernel(x, Wq, K_ext, V_ext, Wo):
    h0 = lax.axis_index("i") * H_LOC
    k_shard = lax.dynamic_slice(K_ext, (0, 0, h0, 0), (B, SKV, H_LOC, DH))
    v_shard = lax.dynamic_slice(V_ext, (0, 0, h0, 0), (B, SKV, H_LOC, DH))

    def body(x_ref, wq_ref, k_ref, v_ref, wo_ref, out_ref,
             cbuf, rbuf, q2_ref, send_sems, recv_sems):
        my = lax.axis_index("i")

        r = my ^ ((my >> 1) & 1)
        r0 = r & 1
        r1 = (r >> 1) & 1
        r2 = (r >> 2) & 1

        def pos_of(q):
            return q ^ ((q >> 1) & 1)

        px = pos_of(r ^ 1)
        py = pos_of(r ^ 2)
        pz = pos_of(r ^ 4)

        barrier_sem = pltpu.get_barrier_semaphore()
        for nbr in (px, py, pz):
            pl.semaphore_signal(barrier_sem, inc=1, device_id=(nbr,),
                                device_id_type=pl.DeviceIdType.MESH)
        pl.semaphore_wait(barrier_sem, 3)

        k1 = SQ * r0
        o1 = SQ * (1 - r0)
        k2 = k1 + 256 * r1
        o2 = k1 + 256 * (1 - r1)
        k3 = k2 + 128 * r2
        o3 = k2 + 128 * (1 - r2)

        def exchange(idx, src_region, dst_is_rbuf, dst_region, partner):
            dst = rbuf if dst_is_rbuf else cbuf
            return pltpu.make_async_remote_copy(
                src_ref=cbuf.at[src_region],
                dst_ref=dst.at[dst_region],
                send_sem=send_sems.at[idx],
                recv_sem=recv_sems.at[idx],
                device_id=(partner,),
                device_id_type=pl.DeviceIdType.MESH,
            )

        xb = x_ref[...].reshape(R, E).astype(jnp.bfloat16)
        wq = wq_ref[...].astype(jnp.bfloat16)
        wo = wo_ref[...].astype(jnp.bfloat16)
        q2_ref[...] = jnp.dot(xb, wq,
                              preferred_element_type=jnp.float32)

        rows = lax.broadcasted_iota(jnp.int32, (SQ, SKV), 0) // BLK
        cols = lax.broadcasted_iota(jnp.int32, (SQ, SKV), 1) // BLK
        live = (rows == cols) | ((cols % 4) == (rows % 4))
        bias = jnp.where(live, jnp.float32(0.0), jnp.float32(-30.0))

        def compute_batch(b):
            qb = q2_ref[pl.ds(b * SQ, SQ), :]
            kb = k_ref[pl.ds(b, 1)][0].astype(jnp.bfloat16)
            vb = v_ref[pl.ds(b, 1)][0].astype(jnp.bfloat16)
            ctx_cols = []
            for h in range(H_LOC):
                q = qb[:, h * DH:(h + 1) * DH].astype(jnp.bfloat16)
                k = kb[:, h, :]
                v = vb[:, h, :]
                s = lax.dot_general(q, k, (((1,), (1,)), ((), ())),
                                    preferred_element_type=jnp.float32)
                s = s * 0.125 + bias
                if os.environ.get("ABLATE") == "nosm":
                    w = s
                    denom = jnp.float32(1.0)
                else:
                    w = jnp.exp(s)
                    denom = jnp.sum(w, axis=1, keepdims=True)
                ctx_h = jnp.dot(w.astype(jnp.bfloat16), v,
                                preferred_element_type=jnp.float32)
                ctx_cols.append(ctx_h / denom)
            ctx = jnp.concatenate(ctx_cols, axis=1).astype(jnp.bfloat16)
            return jnp.dot(ctx, wo, preferred_element_type=jnp.float32)

        _ablate = os.environ.get("ABLATE", "")
        if _ablate == "compute":
            out_ref[pl.ds(o1, SQ)] = compute_batch(1 - r0)
            out_ref[pl.ds(k1, SQ)] = compute_batch(r0)
            return

        if _ablate == "comm":
            part_send = jnp.zeros((SQ, E), jnp.float32)
        else:
            part_send = compute_batch(1 - r0)
        cbuf[pl.ds(o1, SQ)] = part_send.astype(jnp.bfloat16)
        rs1 = exchange(0, pl.ds(o1, SQ), True, pl.ds(0, 512), px)
        rs1.start()

        if _ablate == "comm":
            part_keep = jnp.zeros((SQ, E), jnp.float32)
        else:
            part_keep = compute_batch(r0)
        rs1.wait()
        out_ref[pl.ds(k1, SQ)] = part_keep + rbuf[0:512].astype(jnp.float32)

        cbuf[pl.ds(o2, 256)] = out_ref[pl.ds(o2, 256)].astype(jnp.bfloat16)
        rs2 = exchange(1, pl.ds(o2, 256), True, pl.ds(512, 256), py)
        rs2.start()
        rs2.wait()
        out_ref[pl.ds(k2, 256)] = (out_ref[pl.ds(k2, 256)]
                                   + rbuf[512:768].astype(jnp.float32))

        cbuf[pl.ds(o3, 128)] = out_ref[pl.ds(o3, 128)].astype(jnp.bfloat16)
        rs3 = exchange(2, pl.ds(o3, 128), True, pl.ds(768, 128), pz)
        rs3.start()
        rs3.wait()
        out_ref[pl.ds(k3, 128)] = (out_ref[pl.ds(k3, 128)]
                                   + rbuf[768:896].astype(jnp.float32))

        cbuf[pl.ds(k3, 128)] = out_ref[pl.ds(k3, 128)].astype(jnp.bfloat16)
        ag1 = exchange(3, pl.ds(k3, 128), False, pl.ds(k3, 128), pz)
        ag1.start()
        ag1.wait()

        ag2 = exchange(4, pl.ds(k2, 256), False, pl.ds(k2, 256), py)
        ag2.start()
        ag2.wait()

        ag3 = exchange(5, pl.ds(k1, 512), False, pl.ds(k1, 512), px)
        ag3.start()
        ag3.wait()

        out_ref[pl.ds(o3, 128)] = cbuf[pl.ds(o3, 128)].astype(jnp.float32)
        out_ref[pl.ds(o2, 256)] = cbuf[pl.ds(o2, 256)].astype(jnp.float32)
        out_ref[pl.ds(o1, 512)] = cbuf[pl.ds(o1, 512)].astype(jnp.float32)

    out = pl.pallas_call(
        body,
        out_shape=jax.ShapeDtypeStruct((R, E), jnp.float32),
        in_specs=[pl.BlockSpec(memory_space=pltpu.VMEM)] * 5,
        out_specs=pl.BlockSpec(memory_space=pltpu.VMEM),
        scratch_shapes=[
            pltpu.VMEM((R, E), jnp.bfloat16),
            pltpu.VMEM((896, E), jnp.bfloat16),
            pltpu.VMEM((R, H_LOC * DH), jnp.float32),
            pltpu.SemaphoreType.DMA((6,)),
            pltpu.SemaphoreType.DMA((6,)),
        ],
        compiler_params=pltpu.CompilerParams(collective_id=0),
    )(x, Wq, k_shard, v_shard, Wo)
    return out.reshape(B, SQ, E)


# device time: 48609 ns/iter; 6.9901x vs baseline; 1.2594x over previous
import os

import jax
import jax.numpy as jnp
from jax import lax
from jax.experimental import pallas as pl
from jax.experimental.pallas import tpu as pltpu

N_DEV = 8
B, SQ, SKV, E = 2, 512, 512, 768
HQ, DH = 64, 64
H_LOC = HQ // N_DEV
BLK = 64
R = B * SQ
CH = R // N_DEV


def kernel(x, Wq, K_ext, V_ext, Wo):
    h0 = lax.axis_index("i") * H_LOC
    k_shard = lax.dynamic_slice(K_ext, (0, 0, h0, 0), (B, SKV, H_LOC, DH))
    v_shard = lax.dynamic_slice(V_ext, (0, 0, h0, 0), (B, SKV, H_LOC, DH))

    def body(x_ref, wq_ref, k_ref, v_ref, wo_ref, out_ref,
             cbuf, rbuf, q2_ref, rs_send, rs_recv, ag_send, ag_recv):
        my = lax.axis_index("i")

        barrier_sem = pltpu.get_barrier_semaphore()
        for j in range(N_DEV):
            @pl.when(j != my)
            def _(j=j):
                pl.semaphore_signal(barrier_sem, inc=1, device_id=(j,),
                                    device_id_type=pl.DeviceIdType.MESH)
        pl.semaphore_wait(barrier_sem, N_DEV - 1)

        xb = x_ref[...].reshape(R, E).astype(jnp.bfloat16)
        wq = wq_ref[...].astype(jnp.bfloat16)
        wo = wo_ref[...].astype(jnp.bfloat16)
        q2_ref[...] = jnp.dot(xb, wq, preferred_element_type=jnp.float32)

        rows = lax.broadcasted_iota(jnp.int32, (SQ, SKV), 0) // BLK
        cols = lax.broadcasted_iota(jnp.int32, (SQ, SKV), 1) // BLK
        live = (rows == cols) | ((cols % 4) == (rows % 4))
        bias = jnp.where(live, jnp.float32(0.0), jnp.float32(-30.0))

        def compute_batch(b):
            qb = q2_ref[b * SQ:(b + 1) * SQ, :]
            kb = k_ref[b].astype(jnp.bfloat16)
            vb = v_ref[b].astype(jnp.bfloat16)
            ctx_cols = []
            for h in range(H_LOC):
                q = qb[:, h * DH:(h + 1) * DH].astype(jnp.bfloat16)
                s = lax.dot_general(q, kb[:, h, :], (((1,), (1,)), ((), ())),
                                    preferred_element_type=jnp.float32)
                s = s * 0.125 + bias
                w = jnp.exp(s)
                denom = jnp.sum(w, axis=1, keepdims=True)
                ctx_h = jnp.dot(w.astype(jnp.bfloat16), vb[:, h, :],
                                preferred_element_type=jnp.float32)
                ctx_cols.append(ctx_h / denom)
            ctx = jnp.concatenate(ctx_cols, axis=1).astype(jnp.bfloat16)
            return jnp.dot(ctx, wo, preferred_element_type=jnp.float32)

        def rs_to(c):
            return pltpu.make_async_remote_copy(
                src_ref=cbuf.at[pl.ds(CH * c, CH)],
                dst_ref=rbuf.at[pl.ds(CH * my, CH)],
                send_sem=rs_send.at[c],
                recv_sem=rs_recv.at[my],
                device_id=(c,),
                device_id_type=pl.DeviceIdType.MESH,
            )

        def rs_from(s):
            return pltpu.make_async_remote_copy(
                src_ref=cbuf.at[pl.ds(CH * s, CH)],
                dst_ref=rbuf.at[pl.ds(CH * s, CH)],
                send_sem=rs_send.at[s],
                recv_sem=rs_recv.at[s],
                device_id=(s,),
                device_id_type=pl.DeviceIdType.MESH,
            )

        def ag_to(j):
            return pltpu.make_async_remote_copy(
                src_ref=cbuf.at[pl.ds(CH * my, CH)],
                dst_ref=cbuf.at[pl.ds(CH * my, CH)],
                send_sem=ag_send.at[j],
                recv_sem=ag_recv.at[my],
                device_id=(j,),
                device_id_type=pl.DeviceIdType.MESH,
            )

        def ag_from(s):
            return pltpu.make_async_remote_copy(
                src_ref=cbuf.at[pl.ds(CH * s, CH)],
                dst_ref=cbuf.at[pl.ds(CH * s, CH)],
                send_sem=ag_send.at[s],
                recv_sem=ag_recv.at[s],
                device_id=(s,),
                device_id_type=pl.DeviceIdType.MESH,
            )

        ablate = os.environ.get("ABLATE", "")
        for b in range(B):
            if ablate == "comm":
                part = jnp.zeros((SQ, E), jnp.float32)
            else:
                part = compute_batch(b)
            cbuf[pl.ds(b * SQ, SQ)] = part.astype(jnp.bfloat16)
            out_ref[pl.ds(b * SQ, SQ)] = part
            if ablate != "compute":
                for c in range(b * 4, b * 4 + 4):
                    @pl.when(c != my)
                    def _(c=c):
                        rs_to(c).start()
        if ablate == "compute":
            return

        rbuf[pl.ds(CH * my, CH)] = jnp.zeros((CH, E), jnp.bfloat16)
        for s in range(N_DEV):
            @pl.when(s != my)
            def _(s=s):
                rs_from(s).wait_recv()

        red = (out_ref[pl.ds(CH * my, CH), :]
               + jnp.sum(rbuf[...].reshape(N_DEV, CH, E).astype(jnp.float32),
                         axis=0))
        out_ref[pl.ds(CH * my, CH)] = red
        cbuf[pl.ds(CH * my, CH)] = red.astype(jnp.bfloat16)

        for j in range(N_DEV):
            @pl.when(j != my)
            def _(j=j):
                ag_to(j).start()

        for s in range(N_DEV):
            @pl.when(s != my)
            def _(s=s):
                ag_from(s).wait_recv()
                out_ref[pl.ds(CH * s, CH)] = (
                    cbuf[pl.ds(CH * s, CH)].astype(jnp.float32))

        for c in range(N_DEV):
            @pl.when(c != my)
            def _(c=c):
                rs_to(c).wait_send()
                ag_to(c).wait_send()

    out = pl.pallas_call(
        body,
        out_shape=jax.ShapeDtypeStruct((R, E), jnp.float32),
        in_specs=[pl.BlockSpec(memory_space=pltpu.VMEM)] * 5,
        out_specs=pl.BlockSpec(memory_space=pltpu.VMEM),
        scratch_shapes=[
            pltpu.VMEM((R, E), jnp.bfloat16),
            pltpu.VMEM((R, E), jnp.bfloat16),
            pltpu.VMEM((R, H_LOC * DH), jnp.float32),
            pltpu.SemaphoreType.DMA((N_DEV,)),
            pltpu.SemaphoreType.DMA((N_DEV,)),
            pltpu.SemaphoreType.DMA((N_DEV,)),
            pltpu.SemaphoreType.DMA((N_DEV,)),
        ],
        compiler_params=pltpu.CompilerParams(collective_id=0),
    )(x, Wq, k_shard, v_shard, Wo)
    return out.reshape(B, SQ, E)


# device time: 48481 ns/iter; 7.0086x vs baseline; 1.0026x over previous
import os

import jax
import jax.numpy as jnp
from jax import lax
from jax.experimental import pallas as pl
from jax.experimental.pallas import tpu as pltpu

N_DEV = 8
B, SQ, SKV, E = 2, 512, 512, 768
HQ, DH = 64, 64
H_LOC = HQ // N_DEV
BLK = 64
R = B * SQ
CH = R // N_DEV


def kernel(x, Wq, K_ext, V_ext, Wo):
    h0 = lax.axis_index("i") * H_LOC
    k_shard = lax.dynamic_slice(K_ext, (0, 0, h0, 0), (B, SKV, H_LOC, DH))
    v_shard = lax.dynamic_slice(V_ext, (0, 0, h0, 0), (B, SKV, H_LOC, DH))

    def body(x_ref, wq_ref, k_ref, v_ref, wo_ref, out_ref,
             cbuf, rbuf, rs_send, rs_recv, ag_send, ag_recv):
        my = lax.axis_index("i")

        barrier_sem = pltpu.get_barrier_semaphore()
        for j in range(N_DEV):
            @pl.when(j != my)
            def _(j=j):
                pl.semaphore_signal(barrier_sem, inc=1, device_id=(j,),
                                    device_id_type=pl.DeviceIdType.MESH)
        pl.semaphore_wait(barrier_sem, N_DEV - 1)

        xb = x_ref[...].reshape(R, E).astype(jnp.bfloat16)
        wq = wq_ref[...].astype(jnp.bfloat16)
        wo = wo_ref[...].astype(jnp.bfloat16)
        q2 = jnp.dot(xb, wq, preferred_element_type=jnp.float32)

        rows = lax.broadcasted_iota(jnp.int32, (SQ, SKV), 0) // BLK
        cols = lax.broadcasted_iota(jnp.int32, (SQ, SKV), 1) // BLK
        live = (rows == cols) | ((cols % 4) == (rows % 4))
        bias = jnp.where(live, jnp.float32(0.0), jnp.float32(-30.0))

        def compute_batch(b):
            qb = q2[b * SQ:(b + 1) * SQ, :]
            kb = k_ref[b].astype(jnp.bfloat16)
            vb = v_ref[b].astype(jnp.bfloat16)
            ctx_cols = []
            for h in range(H_LOC):
                q = qb[:, h * DH:(h + 1) * DH].astype(jnp.bfloat16)
                s = lax.dot_general(q, kb[:, h, :], (((1,), (1,)), ((), ())),
                                    preferred_element_type=jnp.float32)
                s = s * 0.125 + bias
                w = jnp.exp(s)
                denom = jnp.sum(w, axis=1, keepdims=True)
                ctx_h = jnp.dot(w.astype(jnp.bfloat16), vb[:, h, :],
                                preferred_element_type=jnp.float32)
                ctx_cols.append(ctx_h / denom)
            ctx = jnp.concatenate(ctx_cols, axis=1).astype(jnp.bfloat16)
            return jnp.dot(ctx, wo, preferred_element_type=jnp.float32)

        def rs_to(c):
            return pltpu.make_async_remote_copy(
                src_ref=cbuf.at[pl.ds(CH * c, CH)],
                dst_ref=rbuf.at[pl.ds(CH * my, CH)],
                send_sem=rs_send.at[c],
                recv_sem=rs_recv.at[my],
                device_id=(c,),
                device_id_type=pl.DeviceIdType.MESH,
            )

        def rs_from(s):
            return pltpu.make_async_remote_copy(
                src_ref=cbuf.at[pl.ds(CH * s, CH)],
                dst_ref=rbuf.at[pl.ds(CH * s, CH)],
                send_sem=rs_send.at[s],
                recv_sem=rs_recv.at[s],
                device_id=(s,),
                device_id_type=pl.DeviceIdType.MESH,
            )

        def ag_to(j):
            return pltpu.make_async_remote_copy(
                src_ref=cbuf.at[pl.ds(CH * my, CH)],
                dst_ref=cbuf.at[pl.ds(CH * my, CH)],
                send_sem=ag_send.at[j],
                recv_sem=ag_recv.at[my],
                device_id=(j,),
                device_id_type=pl.DeviceIdType.MESH,
            )

        def ag_from(s):
            return pltpu.make_async_remote_copy(
                src_ref=cbuf.at[pl.ds(CH * s, CH)],
                dst_ref=cbuf.at[pl.ds(CH * s, CH)],
                send_sem=ag_send.at[s],
                recv_sem=ag_recv.at[s],
                device_id=(s,),
                device_id_type=pl.DeviceIdType.MESH,
            )

        ablate = os.environ.get("ABLATE", "")
        for b in range(B):
            if ablate == "comm":
                part = jnp.zeros((SQ, E), jnp.float32)
            else:
                part = compute_batch(b)
            cbuf[pl.ds(b * SQ, SQ)] = part.astype(jnp.bfloat16)
            if ablate == "compute":
                out_ref[pl.ds(b * SQ, SQ)] = part
            else:
                for c in range(b * 4, b * 4 + 4):
                    @pl.when(c != my)
                    def _(c=c):
                        rs_to(c).start()
        if ablate == "compute":
            return

        rbuf[pl.ds(CH * my, CH)] = cbuf[pl.ds(CH * my, CH)]
        for s in range(N_DEV):
            @pl.when(s != my)
            def _(s=s):
                rs_from(s).wait_recv()

        red = jnp.sum(rbuf[...].reshape(N_DEV, CH, E).astype(jnp.float32),
                      axis=0)
        out_ref[pl.ds(CH * my, CH)] = red
        cbuf[pl.ds(CH * my, CH)] = red.astype(jnp.bfloat16)

        for j in range(N_DEV):
            @pl.when(j != my)
            def _(j=j):
                ag_to(j).start()

        for s in range(N_DEV):
            @pl.when(s != my)
            def _(s=s):
                ag_from(s).wait_recv()
                out_ref[pl.ds(CH * s, CH)] = (
                    cbuf[pl.ds(CH * s, CH)].astype(jnp.float32))

        for c in range(N_DEV):
            @pl.when(c != my)
            def _(c=c):
                rs_to(c).wait_send()
                ag_to(c).wait_send()

    out = pl.pallas_call(
        body,
        out_shape=jax.ShapeDtypeStruct((R, E), jnp.float32),
        in_specs=[pl.BlockSpec(memory_space=pltpu.VMEM)] * 5,
        out_specs=pl.BlockSpec(memory_space=pltpu.VMEM),
        scratch_shapes=[
            pltpu.VMEM((R, E), jnp.bfloat16),
            pltpu.VMEM((R, E), jnp.bfloat16),
            pltpu.SemaphoreType.DMA((N_DEV,)),
            pltpu.SemaphoreType.DMA((N_DEV,)),
            pltpu.SemaphoreType.DMA((N_DEV,)),
            pltpu.SemaphoreType.DMA((N_DEV,)),
        ],
        compiler_params=pltpu.CompilerParams(collective_id=0),
    )(x, Wq, k_shard, v_shard, Wo)
    return out.reshape(B, SQ, E)


# device time: 46544 ns/iter; 7.3003x vs baseline; 1.0416x over previous
import os

import jax
import jax.numpy as jnp
from jax import lax
from jax.experimental import pallas as pl
from jax.experimental.pallas import tpu as pltpu

N_DEV = 8
B, SQ, SKV, E = 2, 512, 512, 768
HQ, DH = 64, 64
H_LOC = HQ // N_DEV
BLK = 64
R = B * SQ
CH = R // N_DEV


def kernel(x, Wq, K_ext, V_ext, Wo):
    h0 = lax.axis_index("i") * H_LOC
    k_shard = lax.dynamic_slice(K_ext, (0, 0, h0, 0), (B, SKV, H_LOC, DH))
    v_shard = lax.dynamic_slice(V_ext, (0, 0, h0, 0), (B, SKV, H_LOC, DH))

    def body(x_ref, wq_ref, k_ref, v_ref, wo_ref, out_ref,
             cbuf, rbuf, rs_send, rs_recv, ag_send, ag_recv):
        my = lax.axis_index("i")

        barrier_sem = pltpu.get_barrier_semaphore()
        for j in range(N_DEV):
            @pl.when(j != my)
            def _(j=j):
                pl.semaphore_signal(barrier_sem, inc=1, device_id=(j,),
                                    device_id_type=pl.DeviceIdType.MESH)
        pl.semaphore_wait(barrier_sem, N_DEV - 1)

        wq = wq_ref[...].astype(jnp.bfloat16)
        wo = wo_ref[...].astype(jnp.bfloat16)

        rows = lax.broadcasted_iota(jnp.int32, (SQ, SKV), 0) // BLK
        cols = lax.broadcasted_iota(jnp.int32, (SQ, SKV), 1) // BLK
        live = (rows == cols) | ((cols % 4) == (rows % 4))
        LOG2E = 1.4426950408889634
        bias = jnp.where(live, jnp.float32(0.0), jnp.float32(-30.0 * LOG2E))

        def compute_batch(b):
            qb = jnp.dot(x_ref[b].astype(jnp.bfloat16), wq,
                         preferred_element_type=jnp.float32
                         ).astype(jnp.bfloat16)
            kb = k_ref[b].astype(jnp.bfloat16)
            vb = v_ref[b].astype(jnp.bfloat16)
            ctx_cols = []
            for h in range(H_LOC):
                q = qb[:, h * DH:(h + 1) * DH]
                s = lax.dot_general(q, kb[:, h, :], (((1,), (1,)), ((), ())),
                                    preferred_element_type=jnp.float32)
                w = jnp.exp2(s * (0.125 * LOG2E) + bias)
                denom = jnp.sum(w, axis=1, keepdims=True)
                ctx_h = jnp.dot(w.astype(jnp.bfloat16), vb[:, h, :],
                                preferred_element_type=jnp.float32)
                ctx_cols.append((ctx_h / denom).astype(jnp.bfloat16))
            ctx = jnp.concatenate(ctx_cols, axis=1)
            return jnp.dot(ctx, wo, preferred_element_type=jnp.float32)

        def rs_to(c):
            return pltpu.make_async_remote_copy(
                src_ref=cbuf.at[pl.ds(CH * c, CH)],
                dst_ref=rbuf.at[pl.ds(CH * my, CH)],
                send_sem=rs_send.at[c],
                recv_sem=rs_recv.at[my],
                device_id=(c,),
                device_id_type=pl.DeviceIdType.MESH,
            )

        def rs_from(s):
            return pltpu.make_async_remote_copy(
                src_ref=cbuf.at[pl.ds(CH * s, CH)],
                dst_ref=rbuf.at[pl.ds(CH * s, CH)],
                send_sem=rs_send.at[s],
                recv_sem=rs_recv.at[s],
                device_id=(s,),
                device_id_type=pl.DeviceIdType.MESH,
            )

        def ag_to(j):
            return pltpu.make_async_remote_copy(
                src_ref=cbuf.at[pl.ds(CH * my, CH)],
                dst_ref=cbuf.at[pl.ds(CH * my, CH)],
                send_sem=ag_send.at[j],
                recv_sem=ag_recv.at[my],
                device_id=(j,),
                device_id_type=pl.DeviceIdType.MESH,
            )

        def ag_from(s):
            return pltpu.make_async_remote_copy(
                src_ref=cbuf.at[pl.ds(CH * s, CH)],
                dst_ref=cbuf.at[pl.ds(CH * s, CH)],
                send_sem=ag_send.at[s],
                recv_sem=ag_recv.at[s],
                device_id=(s,),
                device_id_type=pl.DeviceIdType.MESH,
            )

        ablate = os.environ.get("ABLATE", "")
        for b in range(B):
            if ablate == "comm":
                part = jnp.zeros((SQ, E), jnp.float32)
            else:
                part = compute_batch(b)
            cbuf[pl.ds(b * SQ, SQ)] = part.astype(jnp.bfloat16)
            if ablate == "compute":
                out_ref[pl.ds(b * SQ, SQ)] = part
            else:
                for c in range(b * 4, b * 4 + 4):
                    @pl.when(c != my)
                    def _(c=c):
                        rs_to(c).start()
        if ablate == "compute":
            return

        rbuf[pl.ds(CH * my, CH)] = cbuf[pl.ds(CH * my, CH)]
        for s in range(N_DEV):
            @pl.when(s != my)
            def _(s=s):
                rs_from(s).wait_recv()

        red = jnp.sum(rbuf[...].reshape(N_DEV, CH, E).astype(jnp.float32),
                      axis=0)
        out_ref[pl.ds(CH * my, CH)] = red
        cbuf[pl.ds(CH * my, CH)] = red.astype(jnp.bfloat16)

        for j in range(N_DEV):
            @pl.when(j != my)
            def _(j=j):
                ag_to(j).start()

        for s in range(N_DEV):
            @pl.when(s != my)
            def _(s=s):
                ag_from(s).wait_recv()
                out_ref[pl.ds(CH * s, CH)] = (
                    cbuf[pl.ds(CH * s, CH)].astype(jnp.float32))

        for c in range(N_DEV):
            @pl.when(c != my)
            def _(c=c):
                rs_to(c).wait_send()
                ag_to(c).wait_send()

    out = pl.pallas_call(
        body,
        out_shape=jax.ShapeDtypeStruct((R, E), jnp.float32),
        in_specs=[pl.BlockSpec(memory_space=pltpu.VMEM)] * 5,
        out_specs=pl.BlockSpec(memory_space=pltpu.VMEM),
        scratch_shapes=[
            pltpu.VMEM((R, E), jnp.bfloat16),
            pltpu.VMEM((R, E), jnp.bfloat16),
            pltpu.SemaphoreType.DMA((N_DEV,)),
            pltpu.SemaphoreType.DMA((N_DEV,)),
            pltpu.SemaphoreType.DMA((N_DEV,)),
            pltpu.SemaphoreType.DMA((N_DEV,)),
        ],
        compiler_params=pltpu.CompilerParams(collective_id=0),
    )(x, Wq, k_shard, v_shard, Wo)
    return out.reshape(B, SQ, E)
